# baseline (speedup 1.0000x reference)
import functools

import jax
import jax.numpy as jnp
from jax import lax
from jax.experimental import pallas as pl
from jax.experimental.pallas import tpu as pltpu
from jax.experimental.pallas import tpu_sc as plsc

_EMBED = 64
_FEAT = 128
_BATCH = 16384
_NW = 32
_BPW = _BATCH // _NW
_CHUNK = 128
_NCHUNK = _BPW // _CHUNK
_L = 16
_DEPTH = 8

_MESH = plsc.VectorSubcoreMesh(core_axis_name="c", subcore_axis_name="s")
_PARAMS = pltpu.CompilerParams(needs_layout_passes=False)


def _sc_author(nid_author2d, feats_author):
    @functools.partial(
        pl.kernel,
        mesh=_MESH,
        compiler_params=_PARAMS,
        out_type=jax.ShapeDtypeStruct((_BATCH, _FEAT), jnp.float32),
        scratch_types=[
            pltpu.VMEM((_NCHUNK, _CHUNK), jnp.int32),
            pltpu.VMEM((_BPW, _FEAT), jnp.float32),
            pltpu.SemaphoreType.DMA,
        ],
    )
    def k(nida_hbm, feats_hbm, outa_hbm, idxa_v, rowsa_v, sema):
        wid = lax.axis_index("s") * 2 + lax.axis_index("c")
        base = wid * _BPW
        pltpu.sync_copy(nida_hbm.at[pl.ds(wid * _NCHUNK, _NCHUNK)], idxa_v)
        a_copies = []
        for c in range(_NCHUNK):
            a_copies.append(pltpu.async_copy(
                feats_hbm.at[idxa_v.at[c]],
                rowsa_v.at[pl.ds(c * _CHUNK, _CHUNK)], sema))
        for cp in a_copies:
            cp.wait()
        pltpu.sync_copy(rowsa_v, outa_hbm.at[pl.ds(base, _BPW)])

    return k(nid_author2d, feats_author)


def _sc_paper(nid_paper, emb_paperT):
    @functools.partial(
        pl.kernel,
        mesh=_MESH,
        compiler_params=_PARAMS,
        out_type=jax.ShapeDtypeStruct((_EMBED, _BATCH), jnp.float32),
        scratch_types=[
            pltpu.VMEM((_BPW + 2 * _L,), jnp.int32),
            pltpu.VMEM((_DEPTH * _EMBED, _CHUNK), jnp.float32),
            pltpu.VMEM((_EMBED, _CHUNK), jnp.float32),
            pltpu.SemaphoreType.DMA,
        ],
    )
    def k(nidp_hbm, embT_hbm, outpT_hbm, idxp_v, tbuf_v, poutT_v, semp):
        wid = lax.axis_index("s") * 2 + lax.axis_index("c")
        base = wid * _BPW
        pltpu.sync_copy(nidp_hbm.at[pl.ds(base, _BPW)],
                        idxp_v.at[pl.ds(0, _BPW)])

        featvs = [[lax.iota(jnp.int32, _L) + (_L * c2 + _EMBED * slot)
                   for c2 in range(_EMBED // _L)]
                  for slot in range(_DEPTH)]
        outvs = [lax.iota(jnp.int32, _L) + _L * c2
                 for c2 in range(_EMBED // _L)]

        def fire(r, slot):
            joff = pl.multiple_of(r - lax.rem(r, _CHUNK), _CHUNK)
            for f in range(_EMBED // 8):
                pltpu.async_copy(
                    embT_hbm.at[pl.ds(8 * f, 8), pl.ds(joff, _CHUNK)],
                    tbuf_v.at[pl.ds(_EMBED * slot + 8 * f, 8)],
                    semp)

        def drain(slot):
            pltpu.make_async_copy(
                embT_hbm.at[:, pl.ds(0, _CHUNK)],
                tbuf_v.at[pl.ds(_EMBED * slot, _EMBED)],
                semp).wait()

        def extract(r, slot, row):
            col = jnp.broadcast_to(lax.rem(r, _CHUNK) +
                                   jnp.zeros((), jnp.int32), (_L,))
            rowv = jnp.broadcast_to(row + jnp.zeros((), jnp.int32), (_L,))
            for c2 in range(_EMBED // _L):
                vals = plsc.load_gather(tbuf_v, [featvs[slot][c2], col])
                plsc.store_scatter(poutT_v, [outvs[c2], rowv], vals)

        vec0 = idxp_v[pl.ds(0, _L)]
        for w in range(_DEPTH):
            fire(vec0[w], w)

        def chunk(j, carry):
            veca = idxp_v[pl.ds(_L * j, _L)]
            vecb = idxp_v[pl.ds(_L * j + _L, _L)]
            qrow = (j & (_CHUNK // _L - 1)) * _L
            for b in range(_L):
                slot = b % _DEPTH
                drain(slot)
                extract(veca[b], slot, qrow + b)
                rnext = veca[b + _DEPTH] if b + _DEPTH < _L else (
                    vecb[b + _DEPTH - _L])
                i_next = _L * j + b + _DEPTH

                @pl.when(i_next < _BPW)
                def _():
                    fire(rnext, slot)

            @pl.when((j & (_CHUNK // _L - 1)) == (_CHUNK // _L - 1))
            def _():
                pltpu.sync_copy(
                    poutT_v,
                    outpT_hbm.at[:, pl.ds(
                        base + (j // (_CHUNK // _L)) * _CHUNK, _CHUNK)])
            return carry

        lax.fori_loop(0, _BPW // _L, chunk, 0, unroll=False)

    return k(nid_paper, emb_paperT)


def _tc_matmul_body(x_ref, w_ref, o_ref):
    o_ref[...] = jnp.dot(x_ref[...], w_ref[...],
                         preferred_element_type=jnp.float32)


def _tc_project(x, w):
    rows = 2048
    grid = _BATCH // rows
    return pl.pallas_call(
        _tc_matmul_body,
        grid=(grid,),
        in_specs=[
            pl.BlockSpec((rows, _FEAT), lambda i: (i, 0)),
            pl.BlockSpec((_FEAT, _EMBED), lambda i: (0, 0)),
        ],
        out_specs=pl.BlockSpec((rows, _EMBED), lambda i: (i, 0)),
        out_shape=jax.ShapeDtypeStruct((_BATCH, _EMBED), jnp.float32),
    )(x, w)


def kernel(nid_paper, nid_author, emb_paper, feats_author, W_author):
    nidp = nid_paper.astype(jnp.int32)
    nida = nid_author.astype(jnp.int32).reshape(_NW * _NCHUNK, _CHUNK)
    feats_g = _sc_author(nida, feats_author)
    x_paperT = _sc_paper(nidp, emb_paper.T)
    x_author = _tc_project(feats_g, W_author)
    return (x_paperT.T, x_author)


# speedup vs baseline: 1.0095x; 1.0095x over previous
import functools

import jax
import jax.numpy as jnp
from jax import lax
from jax.experimental import pallas as pl
from jax.experimental.pallas import tpu as pltpu
from jax.experimental.pallas import tpu_sc as plsc

_EMBED = 64
_FEAT = 128
_BATCH = 16384
_NW = 32
_BPW = _BATCH // _NW
_CHUNK = 128
_NCHUNK = _BPW // _CHUNK
_L = 16
_DEPTH = 8

_MESH = plsc.VectorSubcoreMesh(core_axis_name="c", subcore_axis_name="s")
_PARAMS = pltpu.CompilerParams(needs_layout_passes=False)


def _sc_author(nid_author2d, feats_author):
    @functools.partial(
        pl.kernel,
        mesh=_MESH,
        compiler_params=_PARAMS,
        out_type=jax.ShapeDtypeStruct((_BATCH, _FEAT), jnp.float32),
        scratch_types=[
            pltpu.VMEM((_NCHUNK, _CHUNK), jnp.int32),
            pltpu.VMEM((_BPW, _FEAT), jnp.float32),
            pltpu.SemaphoreType.DMA,
        ],
    )
    def k(nida_hbm, feats_hbm, outa_hbm, idxa_v, rowsa_v, sema):
        wid = lax.axis_index("s") * 2 + lax.axis_index("c")
        base = wid * _BPW
        pltpu.sync_copy(nida_hbm.at[pl.ds(wid * _NCHUNK, _NCHUNK)], idxa_v)
        a_copies = []
        for c in range(_NCHUNK):
            a_copies.append(pltpu.async_copy(
                feats_hbm.at[idxa_v.at[c]],
                rowsa_v.at[pl.ds(c * _CHUNK, _CHUNK)], sema))
        for cp in a_copies:
            cp.wait()
        pltpu.sync_copy(rowsa_v, outa_hbm.at[pl.ds(base, _BPW)])

    return k(nid_author2d, feats_author)


def _sc_paper(nid_paper, emb_paperT):
    @functools.partial(
        pl.kernel,
        mesh=_MESH,
        compiler_params=_PARAMS,
        out_type=jax.ShapeDtypeStruct((_EMBED, _BATCH), jnp.float32),
        scratch_types=[
            pltpu.VMEM((_BPW + 2 * _L,), jnp.int32),
            pltpu.VMEM((_DEPTH * _EMBED, _CHUNK), jnp.float32),
            pltpu.VMEM((_EMBED, _CHUNK), jnp.float32),
            pltpu.SemaphoreType.DMA,
        ],
    )
    def k(nidp_hbm, embT_hbm, outpT_hbm, idxp_v, tbuf_v, poutT_v, semp):
        wid = lax.axis_index("s") * 2 + lax.axis_index("c")
        base = wid * _BPW
        pltpu.sync_copy(nidp_hbm.at[pl.ds(base, _BPW)],
                        idxp_v.at[pl.ds(0, _BPW)])

        featvs = [[lax.iota(jnp.int32, _L) + (_L * c2 + _EMBED * slot)
                   for c2 in range(_EMBED // _L)]
                  for slot in range(_DEPTH)]
        outvs = [lax.iota(jnp.int32, _L) + _L * c2
                 for c2 in range(_EMBED // _L)]

        def fire(r, slot):
            joff = pl.multiple_of(r - lax.rem(r, _CHUNK), _CHUNK)
            pltpu.async_copy(
                embT_hbm.at[:, pl.ds(joff, _CHUNK)],
                tbuf_v.at[pl.ds(_EMBED * slot, _EMBED)],
                semp)

        def drain(slot):
            pltpu.make_async_copy(
                embT_hbm.at[:, pl.ds(0, _CHUNK)],
                tbuf_v.at[pl.ds(_EMBED * slot, _EMBED)],
                semp).wait()

        def extract(r, slot, row):
            col = jnp.broadcast_to(lax.rem(r, _CHUNK) +
                                   jnp.zeros((), jnp.int32), (_L,))
            rowv = jnp.broadcast_to(row + jnp.zeros((), jnp.int32), (_L,))
            for c2 in range(_EMBED // _L):
                vals = plsc.load_gather(tbuf_v, [featvs[slot][c2], col])
                plsc.store_scatter(poutT_v, [outvs[c2], rowv], vals)

        vec0 = idxp_v[pl.ds(0, _L)]
        for w in range(_DEPTH):
            fire(vec0[w], w)

        def chunk(j, carry):
            veca = idxp_v[pl.ds(_L * j, _L)]
            vecb = idxp_v[pl.ds(_L * j + _L, _L)]
            qrow = (j & (_CHUNK // _L - 1)) * _L
            for b in range(_L):
                slot = b % _DEPTH
                drain(slot)
                extract(veca[b], slot, qrow + b)
                rnext = veca[b + _DEPTH] if b + _DEPTH < _L else (
                    vecb[b + _DEPTH - _L])
                i_next = _L * j + b + _DEPTH

                @pl.when(i_next < _BPW)
                def _():
                    fire(rnext, slot)

            @pl.when((j & (_CHUNK // _L - 1)) == (_CHUNK // _L - 1))
            def _():
                pltpu.sync_copy(
                    poutT_v,
                    outpT_hbm.at[:, pl.ds(
                        base + (j // (_CHUNK // _L)) * _CHUNK, _CHUNK)])
            return carry

        lax.fori_loop(0, _BPW // _L, chunk, 0, unroll=False)

    return k(nid_paper, emb_paperT)


def _tc_matmul_body(x_ref, w_ref, o_ref):
    o_ref[...] = jnp.dot(x_ref[...], w_ref[...],
                         preferred_element_type=jnp.float32)


def _tc_project(x, w):
    rows = 2048
    grid = _BATCH // rows
    return pl.pallas_call(
        _tc_matmul_body,
        grid=(grid,),
        in_specs=[
            pl.BlockSpec((rows, _FEAT), lambda i: (i, 0)),
            pl.BlockSpec((_FEAT, _EMBED), lambda i: (0, 0)),
        ],
        out_specs=pl.BlockSpec((rows, _EMBED), lambda i: (i, 0)),
        out_shape=jax.ShapeDtypeStruct((_BATCH, _EMBED), jnp.float32),
    )(x, w)


def kernel(nid_paper, nid_author, emb_paper, feats_author, W_author):
    nidp = nid_paper.astype(jnp.int32)
    nida = nid_author.astype(jnp.int32).reshape(_NW * _NCHUNK, _CHUNK)
    feats_g = _sc_author(nida, feats_author)
    x_paperT = _sc_paper(nidp, emb_paper.T)
    x_author = _tc_project(feats_g, W_author)
    return (x_paperT.T, x_author)
